# baseline (speedup 1.0000x reference)
import functools

import jax
import jax.numpy as jnp
from jax import lax
from jax.experimental import pallas as pl
from jax.experimental.pallas import tpu as pltpu
from jax.experimental.pallas import tpu_sc as plsc

N = 10000
E = 320000
D = 128
DX = 144


BN = 1000
GRID = N // BN


def _proj_body(x_ref, w_ref, asrc_ref, adst_ref, hx_ref, ad_ref):
    h = jnp.dot(x_ref[...], w_ref[...], preferred_element_type=jnp.float32)
    hx_ref[:, 0:D] = h
    hx_ref[:, D:D + 1] = jnp.sum(h * asrc_ref[...], axis=1, keepdims=True)
    hx_ref[:, D + 1:DX] = jnp.zeros((BN, DX - D - 1), jnp.float32)
    ad_ref[...] = (jnp.sum(h * adst_ref[...], axis=1, keepdims=True)
                   * jnp.ones((1, 16), jnp.float32))


def _tc_proj(x, W, a_src, a_dst):
    return pl.pallas_call(
        _proj_body,
        grid=(GRID,),
        in_specs=[
            pl.BlockSpec((BN, D), lambda i: (i, 0)),
            pl.BlockSpec((D, D), lambda i: (0, 0)),
            pl.BlockSpec((1, D), lambda i: (0, 0)),
            pl.BlockSpec((1, D), lambda i: (0, 0)),
        ],
        out_specs=[
            pl.BlockSpec((BN, DX), lambda i: (i, 0)),
            pl.BlockSpec((BN, 16), lambda i: (i, 0)),
        ],
        out_shape=[
            jax.ShapeDtypeStruct((N, DX), jnp.float32),
            jax.ShapeDtypeStruct((N, 16), jnp.float32),
        ],
    )(x, W, a_src.reshape(1, D), a_dst.reshape(1, D))


def _mid_body(acc_ref, b_ref, w_ref, asrc_ref, adst_ref, hx_ref, ad_ref):
    num = acc_ref[0, :, 0:D] + acc_ref[1, :, 0:D]
    den = acc_ref[0, :, D:D + 1] + acc_ref[1, :, D:D + 1]
    out = num / (den + 1e-16) + b_ref[...]
    out = jnp.maximum(out, 0.0)
    h = jnp.dot(out, w_ref[...], preferred_element_type=jnp.float32)
    hx_ref[:, 0:D] = h
    hx_ref[:, D:D + 1] = jnp.sum(h * asrc_ref[...], axis=1, keepdims=True)
    hx_ref[:, D + 1:DX] = jnp.zeros((BN, DX - D - 1), jnp.float32)
    ad_ref[...] = (jnp.sum(h * adst_ref[...], axis=1, keepdims=True)
                   * jnp.ones((1, 16), jnp.float32))


def _tc_mid(acc_p, b, W, a_src, a_dst):
    return pl.pallas_call(
        _mid_body,
        grid=(GRID,),
        in_specs=[
            pl.BlockSpec((2, BN, DX), lambda i: (0, i, 0)),
            pl.BlockSpec((1, D), lambda i: (0, 0)),
            pl.BlockSpec((D, D), lambda i: (0, 0)),
            pl.BlockSpec((1, D), lambda i: (0, 0)),
            pl.BlockSpec((1, D), lambda i: (0, 0)),
        ],
        out_specs=[
            pl.BlockSpec((BN, DX), lambda i: (i, 0)),
            pl.BlockSpec((BN, 16), lambda i: (i, 0)),
        ],
        out_shape=[
            jax.ShapeDtypeStruct((N, DX), jnp.float32),
            jax.ShapeDtypeStruct((N, 16), jnp.float32),
        ],
    )(acc_p, b.reshape(1, D), W, a_src.reshape(1, D), a_dst.reshape(1, D))


def _fin_body(acc_ref, b_ref, o_ref):
    num = acc_ref[0, :, 0:D] + acc_ref[1, :, 0:D]
    den = acc_ref[0, :, D:D + 1] + acc_ref[1, :, D:D + 1]
    out = num / (den + 1e-16) + b_ref[...]
    nrm = jnp.sqrt(jnp.sum(out * out, axis=1, keepdims=True))
    o_ref[...] = out / jnp.maximum(nrm, 1e-12)


def _tc_fin(acc_p, b):
    return pl.pallas_call(
        _fin_body,
        grid=(GRID,),
        in_specs=[
            pl.BlockSpec((2, BN, DX), lambda i: (0, i, 0)),
            pl.BlockSpec((1, D), lambda i: (0, 0)),
        ],
        out_specs=pl.BlockSpec((BN, D), lambda i: (i, 0)),
        out_shape=jax.ShapeDtypeStruct((N, D), jnp.float32),
    )(acc_p, b.reshape(1, D))


NC = 2
NS = 16
NW = NC * NS
EW = E // NW
C = 80
NCHUNK = EW // C
NPAD = 10240
ZROWS = NPAD // NS


_GDN = lax.GatherDimensionNumbers(
    offset_dims=(), collapsed_slice_dims=(0,), start_index_map=(0,))


def _lane_bcast(w, c):
    idx = jnp.full((16, 1), c, jnp.int32)
    return lax.gather(w, idx, _GDN, slice_sizes=(1,),
                      mode=lax.GatherScatterMode.PROMISE_IN_BOUNDS)


def _sc_gat(hx, ad, ei, zacc):
    mesh = plsc.VectorSubcoreMesh(core_axis_name="c", subcore_axis_name="s")

    @functools.partial(
        pl.kernel,
        mesh=mesh,
        compiler_params=pltpu.CompilerParams(
            needs_layout_passes=False, use_tc_tiling_on_sc=False),
        out_type=jax.ShapeDtypeStruct((NC, NPAD, DX), jnp.float32),
        scratch_types=[
            pltpu.VMEM((2, C), jnp.int32),
            pltpu.VMEM((2, C), jnp.int32),
            pltpu.VMEM((2, C), jnp.int32),
            pltpu.VMEM((2, C), jnp.int32),
            pltpu.VMEM((C, 16), jnp.float32),
            pltpu.VMEM((C, 16), jnp.float32),
            pltpu.VMEM((C, DX), jnp.float32),
            pltpu.VMEM((C, DX), jnp.float32),
            pltpu.VMEM_SHARED((NPAD, DX), jnp.float32),
            pltpu.SemaphoreType.DMA,
            pltpu.SemaphoreType.DMA,
            pltpu.SemaphoreType.DMA,
            pltpu.SemaphoreType.DMA,
            pltpu.SemaphoreType.DMA,
            pltpu.SemaphoreType.DMA,
            pltpu.SemaphoreType.DMA,
            pltpu.SemaphoreType.DMA,
        ],
    )
    def k(hx_hbm, ad_hbm, ei_hbm, zacc_hbm, acc_out,
          idx0, idx1, idx2, idx3, adg0, adg1, rows0, rows1,
          acc, semi0, semi1, semi2, semi3, semg0, semg1, sems0, sems1):
        cid = lax.axis_index("c")
        sid = lax.axis_index("s")
        wid = sid * NC + cid

        idxs = [(idx0, semi0), (idx1, semi1), (idx2, semi2), (idx3, semi3)]
        sets = [(adg0, rows0, semg0, sems0), (adg1, rows1, semg1, sems1)]

        zbase = sid * ZROWS
        pltpu.async_copy(zacc_hbm, acc.at[pl.ds(zbase, ZROWS)], semg0)
        pltpu.make_async_copy(
            zacc_hbm, acc.at[pl.ds(zbase, ZROWS)], semg0).wait()

        plsc.subcore_barrier()

        ebase = wid * EW
        iota16 = jnp.arange(16, dtype=jnp.int32)
        col0 = jnp.zeros((16,), jnp.int32)
        col128 = jnp.full((16,), D, jnp.int32)

        def start_idx(g, ib):
            idx, semi = ib
            base = ebase + g * C
            pltpu.async_copy(ei_hbm.at[:, pl.ds(base, C)], idx, semi)

        def start_gat(st, ib):
            adg, rows, semg, _ = st
            idx, semi = ib
            pltpu.make_async_copy(
                ei_hbm.at[:, pl.ds(0, C)], idx, semi).wait()
            pltpu.async_copy(ad_hbm.at[idx.at[1]], adg, semg)
            pltpu.async_copy(hx_hbm.at[idx.at[0]], rows, semg)

        def wait_gat(st, ib):
            adg, rows, semg, _ = st
            idx, _ = ib
            pltpu.make_async_copy(ad_hbm.at[idx.at[1]], adg, semg).wait()
            pltpu.make_async_copy(hx_hbm.at[idx.at[0]], rows, semg).wait()

        def compute(st):
            adg, rows, _, _ = st

            def group(j, carry):
                rowj = iota16 + j * 16
                a1 = plsc.load_gather(rows, [rowj, col128])
                a2 = plsc.load_gather(adg, [rowj, col0])
                e = a1 + a2
                e = jnp.where(e >= 0.0, e, 0.2 * e)
                w = jnp.exp(e)
                plsc.store_scatter(rows, [rowj, col128], w)
                for c in range(16):
                    ws = _lane_bcast(w, c)
                    r = j * 16 + c
                    for s in range(D // 16):
                        rows[r, pl.ds(s * 16, 16)] = (
                            rows[r, pl.ds(s * 16, 16)] * ws)
                return carry

            lax.fori_loop(0, C // 16, group, 0)

        def start_scat(st, ib):
            _, rows, _, sems = st
            idx, _ = ib
            pltpu.async_copy(rows, acc.at[idx.at[1]], sems, add=True)

        def wait_scat(st, ib):
            _, rows, _, sems = st
            idx, _ = ib
            pltpu.make_async_copy(rows, acc.at[idx.at[1]], sems).wait()

        start_idx(0, idxs[0])
        start_idx(1, idxs[1])
        start_gat(sets[0], idxs[0])
        start_gat(sets[1], idxs[1])
        start_idx(2, idxs[2])
        start_idx(3, idxs[3])

        def body(kk, carry):
            g = 4 * kk
            sa, sb = sets
            wait_gat(sa, idxs[0])
            compute(sa)
            start_scat(sa, idxs[0])
            wait_gat(sb, idxs[1])
            compute(sb)
            start_scat(sb, idxs[1])
            wait_scat(sa, idxs[0])
            start_gat(sa, idxs[2])
            start_idx(g + 4, idxs[0])
            wait_scat(sb, idxs[1])
            start_gat(sb, idxs[3])

            @pl.when(g + 5 < NCHUNK)
            def _():
                start_idx(g + 5, idxs[1])

            wait_gat(sa, idxs[2])
            compute(sa)
            start_scat(sa, idxs[2])
            wait_gat(sb, idxs[3])
            compute(sb)
            start_scat(sb, idxs[3])
            wait_scat(sa, idxs[2])
            start_gat(sa, idxs[0])

            @pl.when(g + 6 < NCHUNK)
            def _():
                start_idx(g + 6, idxs[2])

            wait_scat(sb, idxs[3])

            @pl.when(g + 5 < NCHUNK)
            def _():
                start_gat(sb, idxs[1])

            @pl.when(g + 7 < NCHUNK)
            def _():
                start_idx(g + 7, idxs[3])

            return carry

        lax.fori_loop(0, NCHUNK // 4, body, 0)

        wait_gat(sets[0], idxs[0])
        compute(sets[0])
        start_scat(sets[0], idxs[0])
        wait_scat(sets[0], idxs[0])

        plsc.subcore_barrier()

        obase = sid * ZROWS
        pltpu.sync_copy(acc.at[pl.ds(obase, ZROWS)],
                        acc_out.at[cid, pl.ds(obase, ZROWS)])

    return k(hx, ad, ei, zacc)


def kernel(x, edge_index, W1, a_src1, a_dst1, b1, W2, a_src2, a_dst2, b2):
    ei = edge_index.astype(jnp.int32)
    zacc = jnp.zeros((ZROWS, DX), jnp.float32)
    hx1, ad1 = _tc_proj(x, W1, a_src1, a_dst1)
    acc1 = _sc_gat(hx1, ad1, ei, zacc)
    hx2, ad2 = _tc_mid(acc1, b1, W2, a_src2, a_dst2)
    acc2 = _sc_gat(hx2, ad2, ei, zacc)
    return _tc_fin(acc2, b2)


# speedup vs baseline: 1.0475x; 1.0475x over previous
import functools

import jax
import jax.numpy as jnp
from jax import lax
from jax.experimental import pallas as pl
from jax.experimental.pallas import tpu as pltpu
from jax.experimental.pallas import tpu_sc as plsc

N = 10000
E = 320000
D = 128


BN = 1000
GRID = N // BN


def _proj_body(x_ref, w_ref, asrc_ref, adst_ref, h_ref, as_ref, ad_ref):
    h = jnp.dot(x_ref[...], w_ref[...], preferred_element_type=jnp.float32)
    h_ref[...] = h
    ones = jnp.ones((1, 16), jnp.float32)
    as_ref[...] = jnp.sum(h * asrc_ref[...], axis=1, keepdims=True) * ones
    ad_ref[...] = jnp.sum(h * adst_ref[...], axis=1, keepdims=True) * ones


def _tc_proj(x, W, a_src, a_dst):
    return pl.pallas_call(
        _proj_body,
        grid=(GRID,),
        in_specs=[
            pl.BlockSpec((BN, D), lambda i: (i, 0)),
            pl.BlockSpec((D, D), lambda i: (0, 0)),
            pl.BlockSpec((1, D), lambda i: (0, 0)),
            pl.BlockSpec((1, D), lambda i: (0, 0)),
        ],
        out_specs=[
            pl.BlockSpec((BN, D), lambda i: (i, 0)),
            pl.BlockSpec((BN, 16), lambda i: (i, 0)),
            pl.BlockSpec((BN, 16), lambda i: (i, 0)),
        ],
        out_shape=[
            jax.ShapeDtypeStruct((N, D), jnp.float32),
            jax.ShapeDtypeStruct((N, 16), jnp.float32),
            jax.ShapeDtypeStruct((N, 16), jnp.float32),
        ],
    )(x, W, a_src.reshape(1, D), a_dst.reshape(1, D))


def _mid_body(nump_ref, denp_ref, b_ref, w_ref, asrc_ref, adst_ref,
              h_ref, as_ref, ad_ref):
    num = nump_ref[0] + nump_ref[1]
    den = denp_ref[0, :, 0:1] + denp_ref[1, :, 0:1]
    out = num / (den + 1e-16) + b_ref[...]
    out = jnp.maximum(out, 0.0)
    h = jnp.dot(out, w_ref[...], preferred_element_type=jnp.float32)
    h_ref[...] = h
    ones = jnp.ones((1, 16), jnp.float32)
    as_ref[...] = jnp.sum(h * asrc_ref[...], axis=1, keepdims=True) * ones
    ad_ref[...] = jnp.sum(h * adst_ref[...], axis=1, keepdims=True) * ones


def _tc_mid(num_p, den_p, b, W, a_src, a_dst):
    return pl.pallas_call(
        _mid_body,
        grid=(GRID,),
        in_specs=[
            pl.BlockSpec((2, BN, D), lambda i: (0, i, 0)),
            pl.BlockSpec((2, BN, 16), lambda i: (0, i, 0)),
            pl.BlockSpec((1, D), lambda i: (0, 0)),
            pl.BlockSpec((D, D), lambda i: (0, 0)),
            pl.BlockSpec((1, D), lambda i: (0, 0)),
            pl.BlockSpec((1, D), lambda i: (0, 0)),
        ],
        out_specs=[
            pl.BlockSpec((BN, D), lambda i: (i, 0)),
            pl.BlockSpec((BN, 16), lambda i: (i, 0)),
            pl.BlockSpec((BN, 16), lambda i: (i, 0)),
        ],
        out_shape=[
            jax.ShapeDtypeStruct((N, D), jnp.float32),
            jax.ShapeDtypeStruct((N, 16), jnp.float32),
            jax.ShapeDtypeStruct((N, 16), jnp.float32),
        ],
    )(num_p, den_p, b.reshape(1, D), W, a_src.reshape(1, D),
      a_dst.reshape(1, D))


def _fin_body(nump_ref, denp_ref, b_ref, o_ref):
    num = nump_ref[0] + nump_ref[1]
    den = denp_ref[0, :, 0:1] + denp_ref[1, :, 0:1]
    out = num / (den + 1e-16) + b_ref[...]
    nrm = jnp.sqrt(jnp.sum(out * out, axis=1, keepdims=True))
    o_ref[...] = out / jnp.maximum(nrm, 1e-12)


def _tc_fin(num_p, den_p, b):
    return pl.pallas_call(
        _fin_body,
        grid=(GRID,),
        in_specs=[
            pl.BlockSpec((2, BN, D), lambda i: (0, i, 0)),
            pl.BlockSpec((2, BN, 16), lambda i: (0, i, 0)),
            pl.BlockSpec((1, D), lambda i: (0, 0)),
        ],
        out_specs=pl.BlockSpec((BN, D), lambda i: (i, 0)),
        out_shape=jax.ShapeDtypeStruct((N, D), jnp.float32),
    )(num_p, den_p, b.reshape(1, D))


NC = 2
NS = 16
NW = NC * NS
EW = E // NW
C = 80
NCHUNK = EW // C
NPAD = 10240
ZROWS = NPAD // NS
OROWS = NPAD // NS


_GDN = lax.GatherDimensionNumbers(
    offset_dims=(), collapsed_slice_dims=(0,), start_index_map=(0,))


def _lane_bcast(w, c):
    idx = jnp.full((16, 1), c, jnp.int32)
    return lax.gather(w, idx, _GDN, slice_sizes=(1,),
                      mode=lax.GatherScatterMode.PROMISE_IN_BOUNDS)


def _sc_gat(h, a_s, a_d, src, dst, znum, zden):
    mesh = plsc.VectorSubcoreMesh(core_axis_name="c", subcore_axis_name="s")

    @functools.partial(
        pl.kernel,
        mesh=mesh,
        compiler_params=pltpu.CompilerParams(
            needs_layout_passes=False, use_tc_tiling_on_sc=False),
        out_type=[
            jax.ShapeDtypeStruct((NC, NPAD, D), jnp.float32),
            jax.ShapeDtypeStruct((NC, NPAD, 16), jnp.float32),
        ],
        scratch_types=[
            pltpu.VMEM((2, C), jnp.int32),
            pltpu.VMEM((2, C), jnp.int32),
            pltpu.VMEM((2, C), jnp.int32),
            pltpu.VMEM((2, C), jnp.int32),
            pltpu.VMEM((C, 16), jnp.float32),
            pltpu.VMEM((C, 16), jnp.float32),
            pltpu.VMEM((C, 16), jnp.float32),
            pltpu.VMEM((C, 16), jnp.float32),
            pltpu.VMEM((C, D), jnp.float32),
            pltpu.VMEM((C, D), jnp.float32),
            pltpu.VMEM((C, 16), jnp.float32),
            pltpu.VMEM((C, 16), jnp.float32),
            pltpu.VMEM_SHARED((NPAD, D), jnp.float32),
            pltpu.VMEM_SHARED((NPAD, 16), jnp.float32),
            pltpu.SemaphoreType.DMA,
            pltpu.SemaphoreType.DMA,
            pltpu.SemaphoreType.DMA,
            pltpu.SemaphoreType.DMA,
            pltpu.SemaphoreType.DMA,
            pltpu.SemaphoreType.DMA,
            pltpu.SemaphoreType.DMA,
            pltpu.SemaphoreType.DMA,
        ],
    )
    def k(h_hbm, as_hbm, ad_hbm, src_hbm, dst_hbm, znum_hbm, zden_hbm,
          num_out, den_out,
          idx0, idx1, idx2, idx3, asg0, asg1, adg0, adg1,
          rows0, rows1, denb0, denb1,
          num_acc, den_acc, semi0, semi1, semi2, semi3,
          semg0, semg1, sems0, sems1):
        cid = lax.axis_index("c")
        sid = lax.axis_index("s")
        wid = sid * NC + cid

        idxs = [(idx0, semi0), (idx1, semi1), (idx2, semi2), (idx3, semi3)]
        sets = [
            (asg0, adg0, rows0, denb0, semg0, sems0),
            (asg1, adg1, rows1, denb1, semg1, sems1),
        ]

        zbase = sid * ZROWS
        pltpu.async_copy(znum_hbm, num_acc.at[pl.ds(zbase, ZROWS)], semg0)
        pltpu.async_copy(zden_hbm, den_acc.at[pl.ds(zbase, ZROWS)], semg0)
        pltpu.make_async_copy(
            znum_hbm, num_acc.at[pl.ds(zbase, ZROWS)], semg0).wait()
        pltpu.make_async_copy(
            zden_hbm, den_acc.at[pl.ds(zbase, ZROWS)], semg0).wait()

        zero16 = jnp.zeros((16,), jnp.float32)
        for r in range(C):
            denb0[r, pl.ds(0, 16)] = zero16
            denb1[r, pl.ds(0, 16)] = zero16

        plsc.subcore_barrier()

        ebase = wid * EW
        iota16 = jnp.arange(16, dtype=jnp.int32)
        col0 = jnp.zeros((16,), jnp.int32)

        def start_idx(g, ib):
            idx, semi = ib
            base = ebase + g * C
            pltpu.async_copy(src_hbm.at[pl.ds(base, C)], idx.at[0], semi)
            pltpu.async_copy(dst_hbm.at[pl.ds(base, C)], idx.at[1], semi)

        def start_gat(st, ib):
            asg, adg, rows, _, semg, _ = st
            idx, semi = ib
            pltpu.make_async_copy(
                src_hbm.at[pl.ds(0, C)], idx.at[0], semi).wait()
            pltpu.make_async_copy(
                dst_hbm.at[pl.ds(0, C)], idx.at[1], semi).wait()
            pltpu.async_copy(as_hbm.at[idx.at[0]], asg, semg)
            pltpu.async_copy(ad_hbm.at[idx.at[1]], adg, semg)
            pltpu.async_copy(h_hbm.at[idx.at[0]], rows, semg)

        def wait_gat(st, ib):
            asg, adg, rows, _, semg, _ = st
            idx, _ = ib
            pltpu.make_async_copy(as_hbm.at[idx.at[0]], asg, semg).wait()
            pltpu.make_async_copy(ad_hbm.at[idx.at[1]], adg, semg).wait()
            pltpu.make_async_copy(h_hbm.at[idx.at[0]], rows, semg).wait()

        def compute(st):
            asg, adg, rows, denb, _, _ = st

            def group(j, carry):
                rowj = iota16 + j * 16
                a1 = plsc.load_gather(asg, [rowj, col0])
                a2 = plsc.load_gather(adg, [rowj, col0])
                e = a1 + a2
                e = jnp.where(e >= 0.0, e, 0.2 * e)
                w = jnp.exp(e)
                plsc.store_scatter(denb, [rowj, col0], w)
                for c in range(16):
                    ws = _lane_bcast(w, c)
                    r = j * 16 + c
                    for s in range(D // 16):
                        rows[r, pl.ds(s * 16, 16)] = (
                            rows[r, pl.ds(s * 16, 16)] * ws)
                return carry

            lax.fori_loop(0, C // 16, group, 0)

        def start_scat(st, ib):
            _, _, rows, denb, _, sems = st
            idx, _ = ib
            pltpu.async_copy(rows, num_acc.at[idx.at[1]], sems, add=True)
            pltpu.async_copy(denb, den_acc.at[idx.at[1]], sems, add=True)

        def wait_scat(st, ib):
            _, _, rows, denb, _, sems = st
            idx, _ = ib
            pltpu.make_async_copy(rows, num_acc.at[idx.at[1]], sems).wait()
            pltpu.make_async_copy(denb, den_acc.at[idx.at[1]], sems).wait()

        start_idx(0, idxs[0])
        start_idx(1, idxs[1])
        start_gat(sets[0], idxs[0])
        start_gat(sets[1], idxs[1])
        start_idx(2, idxs[2])
        start_idx(3, idxs[3])

        def body(kk, carry):
            g = 4 * kk
            sa, sb = sets
            wait_gat(sa, idxs[0])
            compute(sa)
            start_scat(sa, idxs[0])
            wait_gat(sb, idxs[1])
            compute(sb)
            start_scat(sb, idxs[1])
            wait_scat(sa, idxs[0])
            start_gat(sa, idxs[2])
            start_idx(g + 4, idxs[0])
            wait_scat(sb, idxs[1])
            start_gat(sb, idxs[3])

            @pl.when(g + 5 < NCHUNK)
            def _():
                start_idx(g + 5, idxs[1])

            wait_gat(sa, idxs[2])
            compute(sa)
            start_scat(sa, idxs[2])
            wait_gat(sb, idxs[3])
            compute(sb)
            start_scat(sb, idxs[3])
            wait_scat(sa, idxs[2])
            start_gat(sa, idxs[0])

            @pl.when(g + 6 < NCHUNK)
            def _():
                start_idx(g + 6, idxs[2])

            wait_scat(sb, idxs[3])

            @pl.when(g + 5 < NCHUNK)
            def _():
                start_gat(sb, idxs[1])

            @pl.when(g + 7 < NCHUNK)
            def _():
                start_idx(g + 7, idxs[3])

            return carry

        lax.fori_loop(0, NCHUNK // 4, body, 0)

        wait_gat(sets[0], idxs[0])
        compute(sets[0])
        start_scat(sets[0], idxs[0])
        wait_scat(sets[0], idxs[0])

        plsc.subcore_barrier()

        obase = sid * OROWS
        pltpu.sync_copy(num_acc.at[pl.ds(obase, OROWS)],
                        num_out.at[cid, pl.ds(obase, OROWS)])
        pltpu.sync_copy(den_acc.at[pl.ds(obase, OROWS)],
                        den_out.at[cid, pl.ds(obase, OROWS)])

    return k(h, a_s, a_d, src, dst, znum, zden)


def kernel(x, edge_index, W1, a_src1, a_dst1, b1, W2, a_src2, a_dst2, b2):
    src = edge_index[0].astype(jnp.int32)
    dst = edge_index[1].astype(jnp.int32)
    znum = jnp.zeros((ZROWS, D), jnp.float32)
    zden = jnp.zeros((ZROWS, 16), jnp.float32)
    h1, as1, ad1 = _tc_proj(x, W1, a_src1, a_dst1)
    num1, den1 = _sc_gat(h1, as1, ad1, src, dst, znum, zden)
    h2, as2, ad2 = _tc_mid(num1, den1, b1, W2, a_src2, a_dst2)
    num2, den2 = _sc_gat(h2, as2, ad2, src, dst, znum, zden)
    return _tc_fin(num2, den2, b2)


# speedup vs baseline: 1.0532x; 1.0054x over previous
import functools

import jax
import jax.numpy as jnp
from jax import lax
from jax.experimental import pallas as pl
from jax.experimental.pallas import tpu as pltpu
from jax.experimental.pallas import tpu_sc as plsc

N = 10000
E = 320000
D = 128


BN = 1000
GRID = N // BN


def _proj_body(x_ref, w_ref, asrc_ref, adst_ref, h_ref, as_ref, ad_ref):
    h = jnp.dot(x_ref[...], w_ref[...], preferred_element_type=jnp.float32)
    h_ref[...] = h
    ones = jnp.ones((1, 16), jnp.float32)
    as_ref[...] = jnp.sum(h * asrc_ref[...], axis=1, keepdims=True) * ones
    ad_ref[...] = jnp.sum(h * adst_ref[...], axis=1, keepdims=True) * ones


def _tc_proj(x, W, a_src, a_dst):
    return pl.pallas_call(
        _proj_body,
        grid=(GRID,),
        in_specs=[
            pl.BlockSpec((BN, D), lambda i: (i, 0)),
            pl.BlockSpec((D, D), lambda i: (0, 0)),
            pl.BlockSpec((1, D), lambda i: (0, 0)),
            pl.BlockSpec((1, D), lambda i: (0, 0)),
        ],
        out_specs=[
            pl.BlockSpec((BN, D), lambda i: (i, 0)),
            pl.BlockSpec((BN, 16), lambda i: (i, 0)),
            pl.BlockSpec((BN, 16), lambda i: (i, 0)),
        ],
        out_shape=[
            jax.ShapeDtypeStruct((N, D), jnp.float32),
            jax.ShapeDtypeStruct((N, 16), jnp.float32),
            jax.ShapeDtypeStruct((N, 16), jnp.float32),
        ],
    )(x, W, a_src.reshape(1, D), a_dst.reshape(1, D))


def _mid_body(nump_ref, denp_ref, b_ref, w_ref, asrc_ref, adst_ref,
              h_ref, as_ref, ad_ref):
    num = nump_ref[0] + nump_ref[1]
    den = denp_ref[0, :, 0:1] + denp_ref[1, :, 0:1]
    out = num / (den + 1e-16) + b_ref[...]
    out = jnp.maximum(out, 0.0)
    h = jnp.dot(out, w_ref[...], preferred_element_type=jnp.float32)
    h_ref[...] = h
    ones = jnp.ones((1, 16), jnp.float32)
    as_ref[...] = jnp.sum(h * asrc_ref[...], axis=1, keepdims=True) * ones
    ad_ref[...] = jnp.sum(h * adst_ref[...], axis=1, keepdims=True) * ones


def _tc_mid(num_p, den_p, b, W, a_src, a_dst):
    return pl.pallas_call(
        _mid_body,
        grid=(GRID,),
        in_specs=[
            pl.BlockSpec((2, BN, D), lambda i: (0, i, 0)),
            pl.BlockSpec((2, BN, 16), lambda i: (0, i, 0)),
            pl.BlockSpec((1, D), lambda i: (0, 0)),
            pl.BlockSpec((D, D), lambda i: (0, 0)),
            pl.BlockSpec((1, D), lambda i: (0, 0)),
            pl.BlockSpec((1, D), lambda i: (0, 0)),
        ],
        out_specs=[
            pl.BlockSpec((BN, D), lambda i: (i, 0)),
            pl.BlockSpec((BN, 16), lambda i: (i, 0)),
            pl.BlockSpec((BN, 16), lambda i: (i, 0)),
        ],
        out_shape=[
            jax.ShapeDtypeStruct((N, D), jnp.float32),
            jax.ShapeDtypeStruct((N, 16), jnp.float32),
            jax.ShapeDtypeStruct((N, 16), jnp.float32),
        ],
    )(num_p, den_p, b.reshape(1, D), W, a_src.reshape(1, D),
      a_dst.reshape(1, D))


def _fin_body(nump_ref, denp_ref, b_ref, o_ref):
    num = nump_ref[0] + nump_ref[1]
    den = denp_ref[0, :, 0:1] + denp_ref[1, :, 0:1]
    out = num / (den + 1e-16) + b_ref[...]
    nrm = jnp.sqrt(jnp.sum(out * out, axis=1, keepdims=True))
    o_ref[...] = out / jnp.maximum(nrm, 1e-12)


def _tc_fin(num_p, den_p, b):
    return pl.pallas_call(
        _fin_body,
        grid=(GRID,),
        in_specs=[
            pl.BlockSpec((2, BN, D), lambda i: (0, i, 0)),
            pl.BlockSpec((2, BN, 16), lambda i: (0, i, 0)),
            pl.BlockSpec((1, D), lambda i: (0, 0)),
        ],
        out_specs=pl.BlockSpec((BN, D), lambda i: (i, 0)),
        out_shape=jax.ShapeDtypeStruct((N, D), jnp.float32),
    )(num_p, den_p, b.reshape(1, D))


NC = 2
NS = 16
NW = NC * NS
EW = E // NW
C = 80
NCHUNK = EW // C
NPAD = 10240
ZROWS = NPAD // NS
OROWS = NPAD // NS


_GDN = lax.GatherDimensionNumbers(
    offset_dims=(), collapsed_slice_dims=(0,), start_index_map=(0,))


def _lane_bcast(w, c):
    idx = jnp.full((16, 1), c, jnp.int32)
    return lax.gather(w, idx, _GDN, slice_sizes=(1,),
                      mode=lax.GatherScatterMode.PROMISE_IN_BOUNDS)


def _sc_gat(h, a_s, a_d, src, dst, znum, zden):
    mesh = plsc.VectorSubcoreMesh(core_axis_name="c", subcore_axis_name="s")

    @functools.partial(
        pl.kernel,
        mesh=mesh,
        compiler_params=pltpu.CompilerParams(
            needs_layout_passes=False, use_tc_tiling_on_sc=False),
        out_type=[
            jax.ShapeDtypeStruct((NC, NPAD, D), jnp.float32),
            jax.ShapeDtypeStruct((NC, NPAD, 16), jnp.float32),
        ],
        scratch_types=[
            pltpu.VMEM((2, C), jnp.int32),
            pltpu.VMEM((2, C), jnp.int32),
            pltpu.VMEM((2, C), jnp.int32),
            pltpu.VMEM((2, C), jnp.int32),
            pltpu.VMEM((C, 16), jnp.float32),
            pltpu.VMEM((C, 16), jnp.float32),
            pltpu.VMEM((C, 16), jnp.float32),
            pltpu.VMEM((C, 16), jnp.float32),
            pltpu.VMEM((C, D), jnp.float32),
            pltpu.VMEM((C, D), jnp.float32),
            pltpu.VMEM((C, 16), jnp.float32),
            pltpu.VMEM((C, 16), jnp.float32),
            pltpu.VMEM_SHARED((NPAD, D), jnp.float32),
            pltpu.VMEM_SHARED((NPAD, 16), jnp.float32),
            pltpu.SemaphoreType.DMA,
            pltpu.SemaphoreType.DMA,
            pltpu.SemaphoreType.DMA,
            pltpu.SemaphoreType.DMA,
            pltpu.SemaphoreType.DMA,
            pltpu.SemaphoreType.DMA,
            pltpu.SemaphoreType.DMA,
            pltpu.SemaphoreType.DMA,
        ],
    )
    def k(h_hbm, as_hbm, ad_hbm, src_hbm, dst_hbm, znum_hbm, zden_hbm,
          num_out, den_out,
          idx0, idx1, idx2, idx3, asg0, asg1, adg0, adg1,
          rows0, rows1, denb0, denb1,
          num_acc, den_acc, semi0, semi1, semi2, semi3,
          semg0, semg1, sems0, sems1):
        cid = lax.axis_index("c")
        sid = lax.axis_index("s")
        wid = sid * NC + cid

        idxs = [(idx0, semi0), (idx1, semi1), (idx2, semi2), (idx3, semi3)]
        sets = [
            (asg0, adg0, rows0, denb0, semg0, sems0),
            (asg1, adg1, rows1, denb1, semg1, sems1),
        ]

        ebase = wid * EW
        iota16 = jnp.arange(16, dtype=jnp.int32)
        col0 = jnp.zeros((16,), jnp.int32)

        def start_idx(g, ib):
            idx, semi = ib
            base = ebase + g * C
            pltpu.async_copy(src_hbm.at[pl.ds(base, C)], idx.at[0], semi)
            pltpu.async_copy(dst_hbm.at[pl.ds(base, C)], idx.at[1], semi)

        def start_gat(st, ib):
            asg, adg, rows, _, semg, _ = st
            idx, semi = ib
            pltpu.make_async_copy(
                src_hbm.at[pl.ds(0, C)], idx.at[0], semi).wait()
            pltpu.make_async_copy(
                dst_hbm.at[pl.ds(0, C)], idx.at[1], semi).wait()
            pltpu.async_copy(as_hbm.at[idx.at[0]], asg, semg)
            pltpu.async_copy(ad_hbm.at[idx.at[1]], adg, semg)
            pltpu.async_copy(h_hbm.at[idx.at[0]], rows, semg)

        def wait_gat(st, ib):
            asg, adg, rows, _, semg, _ = st
            idx, _ = ib
            pltpu.make_async_copy(as_hbm.at[idx.at[0]], asg, semg).wait()
            pltpu.make_async_copy(ad_hbm.at[idx.at[1]], adg, semg).wait()
            pltpu.make_async_copy(h_hbm.at[idx.at[0]], rows, semg).wait()

        def compute(st):
            asg, adg, rows, denb, _, _ = st

            def group(j, carry):
                rowj = iota16 + j * 16
                a1 = plsc.load_gather(asg, [rowj, col0])
                a2 = plsc.load_gather(adg, [rowj, col0])
                e = a1 + a2
                e = jnp.where(e >= 0.0, e, 0.2 * e)
                w = jnp.exp(e)
                plsc.store_scatter(denb, [rowj, col0], w)
                for c in range(16):
                    ws = _lane_bcast(w, c)
                    r = j * 16 + c
                    for s in range(D // 16):
                        rows[r, pl.ds(s * 16, 16)] = (
                            rows[r, pl.ds(s * 16, 16)] * ws)
                return carry

            lax.fori_loop(0, C // 16, group, 0)

        def start_scat(st, ib):
            _, _, rows, denb, _, sems = st
            idx, _ = ib
            pltpu.async_copy(rows, num_acc.at[idx.at[1]], sems, add=True)
            pltpu.async_copy(denb, den_acc.at[idx.at[1]], sems, add=True)

        def wait_scat(st, ib):
            _, _, rows, denb, _, sems = st
            idx, _ = ib
            pltpu.make_async_copy(rows, num_acc.at[idx.at[1]], sems).wait()
            pltpu.make_async_copy(denb, den_acc.at[idx.at[1]], sems).wait()

        start_idx(0, idxs[0])
        start_idx(1, idxs[1])
        start_gat(sets[0], idxs[0])
        start_gat(sets[1], idxs[1])
        start_idx(2, idxs[2])
        start_idx(3, idxs[3])

        zbase = sid * ZROWS
        pltpu.async_copy(znum_hbm, num_acc.at[pl.ds(zbase, ZROWS)], sems0)
        pltpu.async_copy(zden_hbm, den_acc.at[pl.ds(zbase, ZROWS)], sems0)

        zero16 = jnp.zeros((16,), jnp.float32)
        for r in range(C):
            denb0[r, pl.ds(0, 16)] = zero16
            denb1[r, pl.ds(0, 16)] = zero16

        pltpu.make_async_copy(
            znum_hbm, num_acc.at[pl.ds(zbase, ZROWS)], sems0).wait()
        pltpu.make_async_copy(
            zden_hbm, den_acc.at[pl.ds(zbase, ZROWS)], sems0).wait()

        plsc.subcore_barrier()

        def body(kk, carry):
            g = 4 * kk
            sa, sb = sets
            wait_gat(sa, idxs[0])
            compute(sa)
            start_scat(sa, idxs[0])
            wait_gat(sb, idxs[1])
            compute(sb)
            start_scat(sb, idxs[1])
            wait_scat(sa, idxs[0])
            start_gat(sa, idxs[2])
            start_idx(g + 4, idxs[0])
            wait_scat(sb, idxs[1])
            start_gat(sb, idxs[3])

            @pl.when(g + 5 < NCHUNK)
            def _():
                start_idx(g + 5, idxs[1])

            wait_gat(sa, idxs[2])
            compute(sa)
            start_scat(sa, idxs[2])
            wait_gat(sb, idxs[3])
            compute(sb)
            start_scat(sb, idxs[3])
            wait_scat(sa, idxs[2])
            start_gat(sa, idxs[0])

            @pl.when(g + 6 < NCHUNK)
            def _():
                start_idx(g + 6, idxs[2])

            wait_scat(sb, idxs[3])

            @pl.when(g + 5 < NCHUNK)
            def _():
                start_gat(sb, idxs[1])

            @pl.when(g + 7 < NCHUNK)
            def _():
                start_idx(g + 7, idxs[3])

            return carry

        lax.fori_loop(0, NCHUNK // 4, body, 0)

        wait_gat(sets[0], idxs[0])
        compute(sets[0])
        start_scat(sets[0], idxs[0])
        wait_scat(sets[0], idxs[0])

        plsc.subcore_barrier()

        obase = sid * OROWS
        pltpu.async_copy(num_acc.at[pl.ds(obase, OROWS)],
                         num_out.at[cid, pl.ds(obase, OROWS)], semg0)
        pltpu.async_copy(den_acc.at[pl.ds(obase, OROWS)],
                         den_out.at[cid, pl.ds(obase, OROWS)], semg0)
        pltpu.make_async_copy(num_acc.at[pl.ds(obase, OROWS)],
                              num_out.at[cid, pl.ds(obase, OROWS)],
                              semg0).wait()
        pltpu.make_async_copy(den_acc.at[pl.ds(obase, OROWS)],
                              den_out.at[cid, pl.ds(obase, OROWS)],
                              semg0).wait()

    return k(h, a_s, a_d, src, dst, znum, zden)


def kernel(x, edge_index, W1, a_src1, a_dst1, b1, W2, a_src2, a_dst2, b2):
    src = edge_index[0].astype(jnp.int32)
    dst = edge_index[1].astype(jnp.int32)
    znum = jnp.zeros((ZROWS, D), jnp.float32)
    zden = jnp.zeros((ZROWS, 16), jnp.float32)
    h1, as1, ad1 = _tc_proj(x, W1, a_src1, a_dst1)
    num1, den1 = _sc_gat(h1, as1, ad1, src, dst, znum, zden)
    h2, as2, ad2 = _tc_mid(num1, den1, b1, W2, a_src2, a_dst2)
    num2, den2 = _sc_gat(h2, as2, ad2, src, dst, znum, zden)
    return _tc_fin(num2, den2, b2)
